# baseline (speedup 1.0000x reference)
import jax
import jax.numpy as jnp
from jax.experimental import pallas as pl

_C = 21
_SUB = 8
_LANES = 4096


def _body(pred_ref, gt_ref, out_ref):
    b = pl.program_id(0)
    j = pl.program_id(1)

    x = pred_ref[0]
    best = x[0]
    bidx = jnp.zeros((_SUB, _LANES), jnp.int32)
    for c in range(1, _C):
        xc = x[c]
        upd = xc > best
        best = jnp.where(upd, xc, best)
        bidx = jnp.where(upd, c, bidx)

    gt = gt_ref[0]

    iota = jax.lax.broadcasted_iota(jnp.int32, (_C, _LANES), 0)
    acc = jnp.zeros((_C, _C), jnp.float32)
    for s in range(_SUB):
        ohg = (iota == gt[s : s + 1]).astype(jnp.float32)
        ohp = (iota == bidx[s : s + 1]).astype(jnp.float32)
        acc = acc + jax.lax.dot_general(
            ohg,
            ohp,
            (((1,), (1,)), ((), ())),
            preferred_element_type=jnp.float32,
        )

    @pl.when(jnp.logical_and(b == 0, j == 0))
    def _():
        out_ref[...] = jnp.zeros_like(out_ref)

    out_ref[...] += acc


@jax.jit
def kernel(prediction, groundtruth):
    B = prediction.shape[0]
    npix = prediction.shape[2] * prediction.shape[3]
    rows = npix // _LANES
    grid_j = rows // _SUB

    pred4 = prediction.reshape(B, _C, rows, _LANES)
    gt3 = groundtruth.reshape(B, rows, _LANES)

    out = pl.pallas_call(
        _body,
        grid=(B, grid_j),
        in_specs=[
            pl.BlockSpec((1, _C, _SUB, _LANES), lambda b, j: (b, 0, j, 0)),
            pl.BlockSpec((1, _SUB, _LANES), lambda b, j: (b, j, 0)),
        ],
        out_specs=pl.BlockSpec((_C, _C), lambda b, j: (0, 0)),
        out_shape=jax.ShapeDtypeStruct((_C, _C), jnp.float32),
    )(pred4, gt3)
    return out.astype(jnp.int32)


# speedup vs baseline: 3.3717x; 3.3717x over previous
import jax
import jax.numpy as jnp
from jax.experimental import pallas as pl

_C = 21
_RW = 64
_LN = 512


def _body(pred_ref, gt_ref, out_ref):
    b = pl.program_id(0)
    j = pl.program_id(1)

    x = pred_ref[0]
    best = x[0]
    bidx = jnp.zeros((_RW, _LN), jnp.int32)
    for c in range(1, _C):
        xc = x[c]
        upd = xc > best
        best = jnp.where(upd, xc, best)
        bidx = jnp.where(upd, c, bidx)

    gt = gt_ref[0]

    iota2 = jax.lax.broadcasted_iota(jnp.int32, (_C, _LN), 0)
    acc = jnp.zeros((_C, _C), jnp.float32)
    for r0 in range(0, _RW, 8):
        ohg = jnp.concatenate(
            [(iota2 == gt[r : r + 1]).astype(jnp.float32) for r in range(r0, r0 + 8)],
            axis=1,
        )
        ohp = jnp.concatenate(
            [(iota2 == bidx[r : r + 1]).astype(jnp.float32) for r in range(r0, r0 + 8)],
            axis=1,
        )
        acc = acc + jax.lax.dot_general(
            ohg,
            ohp,
            (((1,), (1,)), ((), ())),
            preferred_element_type=jnp.float32,
        )

    @pl.when(jnp.logical_and(b == 0, j == 0))
    def _():
        out_ref[...] = jnp.zeros_like(out_ref)

    out_ref[...] += acc


@jax.jit
def kernel(prediction, groundtruth):
    B, C, H, W = prediction.shape
    grid_j = H // _RW

    out = pl.pallas_call(
        _body,
        grid=(B, grid_j),
        in_specs=[
            pl.BlockSpec((1, _C, _RW, _LN), lambda b, j: (b, 0, j, 0)),
            pl.BlockSpec((1, _RW, _LN), lambda b, j: (b, j, 0)),
        ],
        out_specs=pl.BlockSpec((_C, _C), lambda b, j: (0, 0)),
        out_shape=jax.ShapeDtypeStruct((_C, _C), jnp.float32),
    )(prediction, groundtruth)
    return out.astype(jnp.int32)


# speedup vs baseline: 4.2810x; 1.2697x over previous
import jax
import jax.numpy as jnp
from jax.experimental import pallas as pl

_C = 21
_RW = 128
_LN = 512


def _body(pred_ref, gt_ref, out_ref):
    b = pl.program_id(0)
    j = pl.program_id(1)

    x = pred_ref[0]
    best = x[0]
    bidx = jnp.zeros((_RW, _LN), jnp.int32)
    for c in range(1, _C):
        xc = x[c]
        upd = xc > best
        best = jnp.where(upd, xc, best)
        bidx = jnp.where(upd, c, bidx)

    gt = gt_ref[0]

    iota2 = jax.lax.broadcasted_iota(jnp.int32, (_C, _LN), 0)
    acc = jnp.zeros((_C, _C), jnp.float32)
    for r0 in range(0, _RW, 8):
        ohg = jnp.concatenate(
            [(iota2 == gt[r : r + 1]).astype(jnp.float32) for r in range(r0, r0 + 8)],
            axis=1,
        )
        ohp = jnp.concatenate(
            [(iota2 == bidx[r : r + 1]).astype(jnp.float32) for r in range(r0, r0 + 8)],
            axis=1,
        )
        acc = acc + jax.lax.dot_general(
            ohg,
            ohp,
            (((1,), (1,)), ((), ())),
            preferred_element_type=jnp.float32,
        )

    @pl.when(jnp.logical_and(b == 0, j == 0))
    def _():
        out_ref[...] = jnp.zeros_like(out_ref)

    out_ref[...] += acc


@jax.jit
def kernel(prediction, groundtruth):
    B, C, H, W = prediction.shape
    grid_j = H // _RW

    out = pl.pallas_call(
        _body,
        grid=(B, grid_j),
        in_specs=[
            pl.BlockSpec((1, _C, _RW, _LN), lambda b, j: (b, 0, j, 0)),
            pl.BlockSpec((1, _RW, _LN), lambda b, j: (b, j, 0)),
        ],
        out_specs=pl.BlockSpec((_C, _C), lambda b, j: (0, 0)),
        out_shape=jax.ShapeDtypeStruct((_C, _C), jnp.float32),
    )(prediction, groundtruth)
    return out.astype(jnp.int32)


# speedup vs baseline: 4.8315x; 1.1286x over previous
import jax
import jax.numpy as jnp
from jax.experimental import pallas as pl

_C = 21
_RW = 256
_LN = 512


def _body(pred_ref, gt_ref, out_ref):
    b = pl.program_id(0)
    j = pl.program_id(1)

    x = pred_ref[0]
    best = x[0]
    bidx = jnp.zeros((_RW, _LN), jnp.int32)
    for c in range(1, _C):
        xc = x[c]
        upd = xc > best
        best = jnp.where(upd, xc, best)
        bidx = jnp.where(upd, c, bidx)

    gt = gt_ref[0]

    iota2 = jax.lax.broadcasted_iota(jnp.int32, (_C, _LN), 0)
    acc = jnp.zeros((_C, _C), jnp.float32)
    for r0 in range(0, _RW, 8):
        ohg = jnp.concatenate(
            [(iota2 == gt[r : r + 1]).astype(jnp.float32) for r in range(r0, r0 + 8)],
            axis=1,
        )
        ohp = jnp.concatenate(
            [(iota2 == bidx[r : r + 1]).astype(jnp.float32) for r in range(r0, r0 + 8)],
            axis=1,
        )
        acc = acc + jax.lax.dot_general(
            ohg,
            ohp,
            (((1,), (1,)), ((), ())),
            preferred_element_type=jnp.float32,
        )

    @pl.when(jnp.logical_and(b == 0, j == 0))
    def _():
        out_ref[...] = jnp.zeros_like(out_ref)

    out_ref[...] += acc


@jax.jit
def kernel(prediction, groundtruth):
    B, C, H, W = prediction.shape
    grid_j = H // _RW

    out = pl.pallas_call(
        _body,
        grid=(B, grid_j),
        in_specs=[
            pl.BlockSpec((1, _C, _RW, _LN), lambda b, j: (b, 0, j, 0)),
            pl.BlockSpec((1, _RW, _LN), lambda b, j: (b, j, 0)),
        ],
        out_specs=pl.BlockSpec((_C, _C), lambda b, j: (0, 0)),
        out_shape=jax.ShapeDtypeStruct((_C, _C), jnp.float32),
    )(prediction, groundtruth)
    return out.astype(jnp.int32)
